# baseline (speedup 1.0000x reference)
import dataclasses
import functools

import jax
import jax.numpy as jnp
from jax import lax
from jax.experimental import pallas as pl
from jax.experimental.pallas import tpu as pltpu
from jax.experimental.pallas import tpu_sc as plsc

N = 10000
D = 128
E = 320000
C = 40

NUM_CORES = 2
NUM_SUBCORES = 16
NUM_WORKERS = NUM_CORES * NUM_SUBCORES
LANES = 16

CHUNK = 120
CHUNKS_A = 84
CHUNKS_B = 84
NCHUNKS = NUM_SUBCORES * (CHUNKS_A + CHUNKS_B)
EP = NCHUNKS * CHUNK
NBUF = 3
IBUF = 6
WB_TILES = 10
WB_ROWS = N // WB_TILES

ROW_BLOCK = 400

_SC_PARAMS = pltpu.CompilerParams()
if "needs_layout_passes" in pltpu.CompilerParams.__dataclass_fields__:
    _SC_PARAMS = dataclasses.replace(_SC_PARAMS, needs_layout_passes=False)


def _sc_aggregate(x, src1d, dst1d, w1d, zeros):
    mesh = plsc.VectorSubcoreMesh(core_axis_name="c", subcore_axis_name="s")

    @functools.partial(
        pl.kernel,
        out_type=jax.ShapeDtypeStruct((NUM_CORES * N, D), jnp.float32),
        mesh=mesh,
        scratch_types=[
            pltpu.VMEM((IBUF, CHUNK), jnp.int32),
            pltpu.VMEM((IBUF, CHUNK), jnp.int32),
            pltpu.VMEM((IBUF, CHUNK), jnp.float32),
            pltpu.VMEM((NBUF, CHUNK, D), jnp.float32),
            pltpu.VMEM_SHARED((N, D), jnp.float32),
            pltpu.SemaphoreType.DMA((NBUF,)),
            pltpu.SemaphoreType.DMA((NBUF,)),
            pltpu.SemaphoreType.DMA((IBUF,)),
            pltpu.SemaphoreType.DMA((IBUF,)),
            pltpu.SemaphoreType.DMA((IBUF,)),
        ],
        compiler_params=_SC_PARAMS,
    )
    def k(x_hbm, src_hbm, dst_hbm, w_hbm, z_hbm, out_hbm,
          si_v, di_v, ww_v, rows_v, acc_sh, gsem, ssem, isem, dsem, wsem):
        c = lax.axis_index("c")
        s = lax.axis_index("s")
        nchunks = jnp.where(c == 0, CHUNKS_A, CHUNKS_B)
        ngroups = nchunks // IBUF
        edge0 = jnp.where(c == 0, s * CHUNKS_A,
                          NUM_SUBCORES * CHUNKS_A + s * CHUNKS_B) * CHUNK

        @pl.when(s < WB_TILES)
        def _():
            pltpu.sync_copy(z_hbm, acc_sh.at[pl.ds(s * WB_ROWS, WB_ROWS)])
        plsc.subcore_barrier()

        def iload(j, m):
            base = edge0 + j * CHUNK
            pltpu.async_copy(src_hbm.at[pl.ds(base, CHUNK)], si_v.at[m],
                             isem.at[m])
            pltpu.async_copy(dst_hbm.at[pl.ds(base, CHUNK)], di_v.at[m],
                             dsem.at[m])
            pltpu.async_copy(w_hbm.at[pl.ds(base, CHUNK)], ww_v.at[m],
                             wsem.at[m])

        def iload_wait(j, m):
            base = edge0 + j * CHUNK
            pltpu.make_async_copy(src_hbm.at[pl.ds(base, CHUNK)], si_v.at[m],
                                  isem.at[m]).wait()
            pltpu.make_async_copy(dst_hbm.at[pl.ds(base, CHUNK)], di_v.at[m],
                                  dsem.at[m]).wait()
            pltpu.make_async_copy(w_hbm.at[pl.ds(base, CHUNK)], ww_v.at[m],
                                  wsem.at[m]).wait()

        def gather(m, u):
            pltpu.async_copy(x_hbm.at[si_v.at[m]], rows_v.at[u], gsem.at[u])

        def gather_wait(m, u):
            pltpu.make_async_copy(x_hbm.at[si_v.at[m]], rows_v.at[u],
                                  gsem.at[u]).wait()

        def scatter(m, u):
            pltpu.async_copy(rows_v.at[u], acc_sh.at[di_v.at[m]], ssem.at[u],
                             add=True)

        def scatter_wait(m, u):
            pltpu.make_async_copy(rows_v.at[u], acc_sh.at[di_v.at[m]],
                                  ssem.at[u]).wait()

        def multiply(m, u):
            cvec = jnp.full((LANES,), m, jnp.int32)

            def edge_body(e, evec):
                wvec = plsc.load_gather(ww_v, [cvec, evec])
                for kk in range(D // LANES):
                    sl = (u, e, pl.ds(kk * LANES, LANES))
                    rows_v[sl] = rows_v[sl] * wvec
                return evec + 1

            lax.fori_loop(0, CHUNK, edge_body,
                          jnp.zeros((LANES,), jnp.int32), unroll=4)

        for t in range(3):
            iload(t, t)
        iload_wait(0, 0)
        gather(0, 0)

        def chunk_body(j, u6):
            u = u6 % NBUF
            un = (u6 + 1) % NBUF
            m = u6
            gather_wait(m, u)

            @pl.when(j >= 2)
            def _():
                scatter_wait((u6 - 2) % IBUF, un)

            @pl.when(j + 3 < nchunks)
            def _():
                iload(j + 3, (u6 + 3) % IBUF)

            @pl.when(j + 1 < nchunks)
            def _():
                iload_wait(j + 1, (u6 + 1) % IBUF)
                gather((u6 + 1) % IBUF, un)

            multiply(m, u)
            scatter(m, u)

        def group_body(g, carry):
            for u6 in range(IBUF):
                chunk_body(IBUF * g + u6, u6)
            return carry

        lax.fori_loop(0, ngroups, group_body, 0)
        for t in (4, 5):
            scatter_wait(t, t % NBUF)

        plsc.subcore_barrier()

        @pl.when(s < WB_TILES)
        def _():
            pltpu.sync_copy(
                acc_sh.at[pl.ds(s * WB_ROWS, WB_ROWS)],
                out_hbm.at[pl.ds(c * N + s * WB_ROWS, WB_ROWS)],
            )

    return k(x, src1d, dst1d, w1d, zeros)


def _dot(a, b):
    return lax.dot_general(a, b, (((1,), (0,)), ((), ())),
                           precision=lax.Precision.HIGHEST,
                           preferred_element_type=jnp.float32)


def _full_spec():
    return pl.BlockSpec((D, D), lambda i: (0, 0))


def _row_spec():
    return pl.BlockSpec((ROW_BLOCK, D), lambda i: (i, 0))


def _matmul_kernel(x_ref, w_ref, o_ref):
    o_ref[...] = _dot(x_ref[...], w_ref[...])


def _tc_matmul(x, w):
    return pl.pallas_call(
        _matmul_kernel,
        grid=(N // ROW_BLOCK,),
        in_specs=[_row_spec(), _full_spec()],
        out_specs=_row_spec(),
        out_shape=jax.ShapeDtypeStruct((N, D), jnp.float32),
    )(x, w)


def _matmul2_kernel(x_ref, wa_ref, wb_ref, oa_ref, ob_ref):
    x = x_ref[...]
    oa_ref[...] = _dot(x, wa_ref[...])
    ob_ref[...] = _dot(x, wb_ref[...])


def _tc_matmul2(x, wa, wb):
    return pl.pallas_call(
        _matmul2_kernel,
        grid=(N // ROW_BLOCK,),
        in_specs=[_row_spec(), _full_spec(), _full_spec()],
        out_specs=[_row_spec(), _row_spec()],
        out_shape=[jax.ShapeDtypeStruct((N, D), jnp.float32)] * 2,
    )(x, wa, wb)


def _combine_kernel(p_ref, r_ref, wrel_ref, b_ref, o_ref):
    agg = p_ref[0] + p_ref[1]
    out = _dot(agg, wrel_ref[...]) + r_ref[...] + b_ref[...]
    o_ref[...] = jnp.maximum(out, 0.0)


def _tc_combine(p, root, wrel, b):
    return pl.pallas_call(
        _combine_kernel,
        grid=(N // ROW_BLOCK,),
        in_specs=[
            pl.BlockSpec((NUM_CORES, ROW_BLOCK, D), lambda i: (0, i, 0)),
            _row_spec(),
            _full_spec(),
            pl.BlockSpec((1, D), lambda i: (0, 0)),
        ],
        out_specs=_row_spec(),
        out_shape=jax.ShapeDtypeStruct((N, D), jnp.float32),
    )(p, root, wrel, b)


def _final_kernel(p_ref, r_ref, l1_ref, wrel_ref, b_ref, wlb_ref, bl_ref,
                  o_ref):
    agg = p_ref[0] + p_ref[1]
    x2 = jnp.maximum(_dot(agg, wrel_ref[...]) + r_ref[...] + b_ref[...], 0.0)
    logits = l1_ref[...] + _dot(x2, wlb_ref[...]) + bl_ref[...]
    mask = lax.broadcasted_iota(jnp.int32, logits.shape, 1) < C
    masked = jnp.where(mask, logits, jnp.float32(-1e30))
    m = jnp.max(masked, axis=-1, keepdims=True)
    z = jnp.where(mask, jnp.exp(logits - m), 0.0)
    lse = jnp.log(jnp.sum(z, axis=-1, keepdims=True)) + m
    o_ref[...] = logits - lse


def _tc_final(p, root, l1, wrel, b, wlb, bl):
    return pl.pallas_call(
        _final_kernel,
        grid=(N // ROW_BLOCK,),
        in_specs=[
            pl.BlockSpec((NUM_CORES, ROW_BLOCK, D), lambda i: (0, i, 0)),
            _row_spec(),
            _row_spec(),
            _full_spec(),
            pl.BlockSpec((1, D), lambda i: (0, 0)),
            _full_spec(),
            pl.BlockSpec((1, D), lambda i: (0, 0)),
        ],
        out_specs=_row_spec(),
        out_shape=jax.ShapeDtypeStruct((N, D), jnp.float32),
    )(p, root, l1, wrel, b, wlb, bl)


def kernel(x0, edge_index, edge_weight, W1rel, b1, W1root, W2rel, b2, W2root,
           Wlin, blin):
    pad = EP - E
    spread = (jnp.arange(pad, dtype=jnp.int32) * 4) % N
    src1d = jnp.concatenate([edge_index[0], spread])
    dst1d = jnp.concatenate([edge_index[1], spread])
    edge_weight = jnp.pad(edge_weight, (0, pad))
    zeros = jnp.zeros((WB_ROWS, D), jnp.float32)

    wla = jnp.pad(Wlin[:D], ((0, 0), (0, D - C)))
    wlb = jnp.pad(Wlin[D:], ((0, 0), (0, D - C)))
    bl = jnp.pad(blin, (0, D - C)).reshape(1, D)

    p1 = _sc_aggregate(x0, src1d, dst1d, edge_weight, zeros)
    root1 = _tc_matmul(x0, W1root)
    x1 = _tc_combine(p1.reshape(NUM_CORES, N, D), root1, W1rel,
                     b1.reshape(1, D))

    p2 = _sc_aggregate(x1, src1d, dst1d, edge_weight, zeros)
    root2, l1 = _tc_matmul2(x1, W2root, wla)
    out_pad = _tc_final(p2.reshape(NUM_CORES, N, D), root2, l1, W2rel,
                        b2.reshape(1, D), wlb, bl)
    return out_pad[:, :C]


# speedup vs baseline: 1.0323x; 1.0323x over previous
import dataclasses
import functools

import jax
import jax.numpy as jnp
from jax import lax
from jax.experimental import pallas as pl
from jax.experimental.pallas import tpu as pltpu
from jax.experimental.pallas import tpu_sc as plsc

N = 10000
D = 128
E = 320000
C = 40

NUM_CORES = 2
NUM_SUBCORES = 16
NUM_WORKERS = NUM_CORES * NUM_SUBCORES
LANES = 16

CHUNK = 120
CHUNKS_A = 84
CHUNKS_B = 84
NCHUNKS = NUM_SUBCORES * (CHUNKS_A + CHUNKS_B)
EP = NCHUNKS * CHUNK
NBUF = 3
IBUF = 6
WB_TILES = 10
WB_ROWS = N // WB_TILES

ROW_BLOCK = 400

_SC_PARAMS = pltpu.CompilerParams()
if "needs_layout_passes" in pltpu.CompilerParams.__dataclass_fields__:
    _SC_PARAMS = dataclasses.replace(_SC_PARAMS, needs_layout_passes=False)


def _sc_aggregate(x, src1d, dst1d, w1d, zeros):
    mesh = plsc.VectorSubcoreMesh(core_axis_name="c", subcore_axis_name="s")

    @functools.partial(
        pl.kernel,
        out_type=jax.ShapeDtypeStruct((NUM_CORES * N, D), jnp.float32),
        mesh=mesh,
        scratch_types=[
            pltpu.VMEM((IBUF, CHUNK), jnp.int32),
            pltpu.VMEM((IBUF, CHUNK), jnp.int32),
            pltpu.VMEM((IBUF, CHUNK), jnp.float32),
            pltpu.VMEM((NBUF, CHUNK, D), jnp.float32),
            pltpu.VMEM_SHARED((N, D), jnp.float32),
            pltpu.SemaphoreType.DMA((NBUF,)),
            pltpu.SemaphoreType.DMA((NBUF,)),
            pltpu.SemaphoreType.DMA((IBUF,)),
            pltpu.SemaphoreType.DMA((IBUF,)),
            pltpu.SemaphoreType.DMA((IBUF,)),
        ],
        compiler_params=_SC_PARAMS,
    )
    def k(x_hbm, src_hbm, dst_hbm, w_hbm, z_hbm, out_hbm,
          si_v, di_v, ww_v, rows_v, acc_sh, gsem, ssem, isem, dsem, wsem):
        c = lax.axis_index("c")
        s = lax.axis_index("s")
        nchunks = jnp.where(c == 0, CHUNKS_A, CHUNKS_B)
        ngroups = nchunks // IBUF
        edge0 = jnp.where(c == 0, s * CHUNKS_A,
                          NUM_SUBCORES * CHUNKS_A + s * CHUNKS_B) * CHUNK

        @pl.when(s < WB_TILES)
        def _():
            pltpu.sync_copy(z_hbm, acc_sh.at[pl.ds(s * WB_ROWS, WB_ROWS)])
        plsc.subcore_barrier()

        def iload(j, m):
            base = edge0 + j * CHUNK
            pltpu.async_copy(src_hbm.at[pl.ds(base, CHUNK)], si_v.at[m],
                             isem.at[m])
            pltpu.async_copy(dst_hbm.at[pl.ds(base, CHUNK)], di_v.at[m],
                             dsem.at[m])
            pltpu.async_copy(w_hbm.at[pl.ds(base, CHUNK)], ww_v.at[m],
                             wsem.at[m])

        def iload_wait(j, m):
            base = edge0 + j * CHUNK
            pltpu.make_async_copy(src_hbm.at[pl.ds(base, CHUNK)], si_v.at[m],
                                  isem.at[m]).wait()
            pltpu.make_async_copy(dst_hbm.at[pl.ds(base, CHUNK)], di_v.at[m],
                                  dsem.at[m]).wait()
            pltpu.make_async_copy(w_hbm.at[pl.ds(base, CHUNK)], ww_v.at[m],
                                  wsem.at[m]).wait()

        def gather(m, u):
            pltpu.async_copy(x_hbm.at[si_v.at[m]], rows_v.at[u], gsem.at[u])

        def gather_wait(m, u):
            pltpu.make_async_copy(x_hbm.at[si_v.at[m]], rows_v.at[u],
                                  gsem.at[u]).wait()

        def scatter(m, u):
            pltpu.async_copy(rows_v.at[u], acc_sh.at[di_v.at[m]], ssem.at[u],
                             add=True)

        def scatter_wait(m, u):
            pltpu.make_async_copy(rows_v.at[u], acc_sh.at[di_v.at[m]],
                                  ssem.at[u]).wait()

        def multiply(m, u):
            cvec = jnp.full((LANES,), m, jnp.int32)

            def edge_body(e, carry):
                wvec = plsc.load_gather(
                    ww_v, [cvec, jnp.full((LANES,), e, jnp.int32)])
                for kk in range(D // LANES):
                    sl = (u, e, pl.ds(kk * LANES, LANES))
                    rows_v[sl] = rows_v[sl] * wvec
                return carry

            lax.fori_loop(0, CHUNK, edge_body, 0, unroll=2)

        for t in range(3):
            iload(t, t)
        iload_wait(0, 0)
        gather(0, 0)

        def chunk_body(j, u6):
            u = u6 % NBUF
            un = (u6 + 1) % NBUF
            m = u6
            gather_wait(m, u)

            @pl.when(j >= 2)
            def _():
                scatter_wait((u6 - 2) % IBUF, un)

            @pl.when(j + 3 < nchunks)
            def _():
                iload(j + 3, (u6 + 3) % IBUF)

            @pl.when(j + 1 < nchunks)
            def _():
                iload_wait(j + 1, (u6 + 1) % IBUF)
                gather((u6 + 1) % IBUF, un)

            multiply(m, u)
            scatter(m, u)

        def group_body(g, carry):
            for u6 in range(IBUF):
                chunk_body(IBUF * g + u6, u6)
            return carry

        lax.fori_loop(0, ngroups, group_body, 0)
        for t in (4, 5):
            scatter_wait(t, t % NBUF)

        plsc.subcore_barrier()

        @pl.when(s < WB_TILES)
        def _():
            pltpu.sync_copy(
                acc_sh.at[pl.ds(s * WB_ROWS, WB_ROWS)],
                out_hbm.at[pl.ds(c * N + s * WB_ROWS, WB_ROWS)],
            )

    return k(x, src1d, dst1d, w1d, zeros)


def _dot(a, b, precision=lax.Precision.DEFAULT):
    return lax.dot_general(a, b, (((1,), (0,)), ((), ())),
                           precision=precision,
                           preferred_element_type=jnp.float32)


def _full_spec():
    return pl.BlockSpec((D, D), lambda i: (0, 0))


def _row_spec():
    return pl.BlockSpec((ROW_BLOCK, D), lambda i: (i, 0))


def _matmul_kernel(x_ref, w_ref, o_ref):
    o_ref[...] = _dot(x_ref[...], w_ref[...])


def _tc_matmul(x, w):
    return pl.pallas_call(
        _matmul_kernel,
        grid=(N // ROW_BLOCK,),
        in_specs=[_row_spec(), _full_spec()],
        out_specs=_row_spec(),
        out_shape=jax.ShapeDtypeStruct((N, D), jnp.float32),
    )(x, w)


def _matmul2_kernel(x_ref, wa_ref, wb_ref, oa_ref, ob_ref):
    x = x_ref[...]
    oa_ref[...] = _dot(x, wa_ref[...])
    ob_ref[...] = _dot(x, wb_ref[...])


def _tc_matmul2(x, wa, wb):
    return pl.pallas_call(
        _matmul2_kernel,
        grid=(N // ROW_BLOCK,),
        in_specs=[_row_spec(), _full_spec(), _full_spec()],
        out_specs=[_row_spec(), _row_spec()],
        out_shape=[jax.ShapeDtypeStruct((N, D), jnp.float32)] * 2,
    )(x, wa, wb)


def _combine_kernel(p_ref, r_ref, wrel_ref, b_ref, o_ref):
    agg = p_ref[0] + p_ref[1]
    out = _dot(agg, wrel_ref[...]) + r_ref[...] + b_ref[...]
    o_ref[...] = jnp.maximum(out, 0.0)


def _tc_combine(p, root, wrel, b):
    return pl.pallas_call(
        _combine_kernel,
        grid=(N // ROW_BLOCK,),
        in_specs=[
            pl.BlockSpec((NUM_CORES, ROW_BLOCK, D), lambda i: (0, i, 0)),
            _row_spec(),
            _full_spec(),
            pl.BlockSpec((1, D), lambda i: (0, 0)),
        ],
        out_specs=_row_spec(),
        out_shape=jax.ShapeDtypeStruct((N, D), jnp.float32),
    )(p, root, wrel, b)


def _final_kernel(p_ref, r_ref, l1_ref, wrel_ref, b_ref, wlb_ref, bl_ref,
                  o_ref):
    agg = p_ref[0] + p_ref[1]
    x2 = jnp.maximum(_dot(agg, wrel_ref[...]) + r_ref[...] + b_ref[...], 0.0)
    logits = l1_ref[...] + _dot(x2, wlb_ref[...]) + bl_ref[...]
    mask = lax.broadcasted_iota(jnp.int32, logits.shape, 1) < C
    masked = jnp.where(mask, logits, jnp.float32(-1e30))
    m = jnp.max(masked, axis=-1, keepdims=True)
    z = jnp.where(mask, jnp.exp(logits - m), 0.0)
    lse = jnp.log(jnp.sum(z, axis=-1, keepdims=True)) + m
    o_ref[...] = logits - lse


def _tc_final(p, root, l1, wrel, b, wlb, bl):
    return pl.pallas_call(
        _final_kernel,
        grid=(N // ROW_BLOCK,),
        in_specs=[
            pl.BlockSpec((NUM_CORES, ROW_BLOCK, D), lambda i: (0, i, 0)),
            _row_spec(),
            _row_spec(),
            _full_spec(),
            pl.BlockSpec((1, D), lambda i: (0, 0)),
            _full_spec(),
            pl.BlockSpec((1, D), lambda i: (0, 0)),
        ],
        out_specs=_row_spec(),
        out_shape=jax.ShapeDtypeStruct((N, D), jnp.float32),
    )(p, root, l1, wrel, b, wlb, bl)


def kernel(x0, edge_index, edge_weight, W1rel, b1, W1root, W2rel, b2, W2root,
           Wlin, blin):
    pad = EP - E
    spread = (jnp.arange(pad, dtype=jnp.int32) * 4) % N
    src1d = jnp.concatenate([edge_index[0], spread])
    dst1d = jnp.concatenate([edge_index[1], spread])
    edge_weight = jnp.pad(edge_weight, (0, pad))
    zeros = jnp.zeros((WB_ROWS, D), jnp.float32)

    wla = jnp.pad(Wlin[:D], ((0, 0), (0, D - C)))
    wlb = jnp.pad(Wlin[D:], ((0, 0), (0, D - C)))
    bl = jnp.pad(blin, (0, D - C)).reshape(1, D)

    p1 = _sc_aggregate(x0, src1d, dst1d, edge_weight, zeros)
    root1 = _tc_matmul(x0, W1root)
    x1 = _tc_combine(p1.reshape(NUM_CORES, N, D), root1, W1rel,
                     b1.reshape(1, D))

    p2 = _sc_aggregate(x1, src1d, dst1d, edge_weight, zeros)
    root2, l1 = _tc_matmul2(x1, W2root, wla)
    out_pad = _tc_final(p2.reshape(NUM_CORES, N, D), root2, l1, W2rel,
                        b2.reshape(1, D), wlb, bl)
    return out_pad[:, :C]
